# Initial kernel scaffold; baseline (speedup 1.0000x reference)
import jax
import jax.numpy as jnp
from jax.experimental import pallas as pl


def kernel(atom, bond, adj_matrix, W_A, b_A, W_B, b_B, gamma, beta, W_conv, b_conv, W_lin, b_lin):
    raise NotImplementedError("write your pallas kernel here")


# speedup vs baseline: 16.4930x; 16.4930x over previous
import jax
import jax.numpy as jnp
from jax.experimental import pallas as pl
from jax.experimental.pallas import tpu as pltpu

B, N, M = 1024, 64, 32
F_IN, F_OUT, F_BOND = 128, 25, 16
HALF = N // 2
C = 32
G1 = 64
P1 = B // G1
E = 2 * N * M
T = 2 * N


def _softplus(x):
    return jnp.logaddexp(x, 0.0)


def _phase1_kernel(atom_ref, wa_ref, ba_ref, wb_ref, bb_ref, h_ref, st_ref):
    x = atom_ref[...]
    x0 = x[:, :HALF, :].reshape(G1 * HALF, F_IN)
    x1 = x[:, HALF:, :].reshape(G1 * HALF, F_IN)
    h0 = _softplus(jnp.dot(x0, wa_ref[...], preferred_element_type=jnp.float32)
                   + ba_ref[...])
    h1 = _softplus(jnp.dot(x1, wb_ref[...], preferred_element_type=jnp.float32)
                   + bb_ref[...])
    h_ref[...] = jnp.concatenate(
        [h0.reshape(G1, HALF, C), h1.reshape(G1, HALF, C)], axis=1)
    s = (jnp.sum(h0, axis=0) + jnp.sum(h1, axis=0)).reshape(1, 1, C)
    sq = (jnp.sum(h0 * h0, axis=0) + jnp.sum(h1 * h1, axis=0)).reshape(1, 1, C)
    st_ref[...] = jnp.concatenate([s, sq], axis=1)


def _phase2_kernel(h_ref, adj_ref, bond_ref, a_ref, c_ref, ws_ref, wn_ref,
                   wb_ref, bc_ref, wl_ref, bl_ref, out_ref):
    hn = h_ref[...].reshape(T, C) * a_ref[...] + c_ref[...]
    s = jnp.dot(hn, ws_ref[...], preferred_element_type=jnp.float32) \
        + bc_ref[...]
    t = jnp.dot(hn, wn_ref[...], preferred_element_type=jnp.float32)
    idx = jnp.transpose(adj_ref[...].reshape(1, E))
    onehot = (idx == jax.lax.broadcasted_iota(jnp.int32, (E, T), 1)
              ).astype(jnp.float32)
    nbr = jnp.dot(onehot, t, preferred_element_type=jnp.float32)
    zb = jnp.dot(bond_ref[...].reshape(E, F_BOND), wb_ref[...],
                 preferred_element_type=jnp.float32)
    se = jnp.broadcast_to(s.reshape(T, 1, 128), (T, M, 128)).reshape(E, 128)
    zw = se + nbr + zb
    gate = jax.nn.sigmoid(zw[:, :64])
    core = _softplus(zw[:, 64:])
    msg = jnp.sum((gate * core).reshape(T, M, 64), axis=1)
    hn64 = jnp.concatenate([hn, jnp.zeros((T, 64 - C), jnp.float32)], axis=1)
    au = _softplus(hn64 + msg)
    pooled = jnp.mean(au.reshape(2, N, 64), axis=1)
    e = jnp.maximum(
        jnp.dot(pooled, wl_ref[...], preferred_element_type=jnp.float32)
        + bl_ref[...], 0.0)
    out_ref[...] = e.reshape(1, 2, 1)


def kernel(atom, bond, adj_matrix, W_A, b_A, W_B, b_B, gamma, beta,
           W_conv, b_conv, W_lin, b_lin):
    f32 = jnp.float32

    wa = jnp.zeros((F_IN, C), f32).at[:, :F_OUT].set(W_A)
    wb = jnp.zeros((F_IN, C), f32).at[:, :F_OUT].set(W_B)
    ba = jnp.zeros((1, C), f32).at[0, :F_OUT].set(b_A)
    bb = jnp.zeros((1, C), f32).at[0, :F_OUT].set(b_B)

    h, stats = pl.pallas_call(
        _phase1_kernel,
        grid=(P1,),
        in_specs=[
            pl.BlockSpec((G1, N, F_IN), lambda i: (i, 0, 0)),
            pl.BlockSpec((F_IN, C), lambda i: (0, 0)),
            pl.BlockSpec((1, C), lambda i: (0, 0)),
            pl.BlockSpec((F_IN, C), lambda i: (0, 0)),
            pl.BlockSpec((1, C), lambda i: (0, 0)),
        ],
        out_specs=[
            pl.BlockSpec((G1, N, C), lambda i: (i, 0, 0)),
            pl.BlockSpec((1, 2, C), lambda i: (i, 0, 0)),
        ],
        out_shape=[
            jax.ShapeDtypeStruct((B, N, C), f32),
            jax.ShapeDtypeStruct((P1, 2, C), f32),
        ],
        compiler_params=pltpu.CompilerParams(
            dimension_semantics=("parallel",)),
    )(atom, wa, ba, wb, bb)

    cnt = float(B * N)
    mean = jnp.sum(stats[:, 0, :], axis=0) / cnt
    var = jnp.sum(stats[:, 1, :], axis=0) / cnt - mean * mean
    g32 = jnp.zeros((C,), f32).at[:F_OUT].set(gamma)
    be32 = jnp.zeros((C,), f32).at[:F_OUT].set(beta)
    a = (g32 * jax.lax.rsqrt(var + 1e-5)).reshape(1, C)
    c = (be32 - mean * a[0]).reshape(1, C)

    def pack(wx, rows):
        out = jnp.zeros((rows, 128), f32)
        out = out.at[:wx.shape[0], 0:F_OUT].set(wx[:, :F_OUT])
        return out.at[:wx.shape[0], 64:64 + F_OUT].set(wx[:, F_OUT:])

    ws = pack(W_conv[0:F_OUT], C)
    wn = pack(W_conv[F_OUT:2 * F_OUT], C)
    wbond = pack(W_conv[2 * F_OUT:], F_BOND)
    bc = pack(b_conv.reshape(1, 2 * F_OUT), 1)
    wl = jnp.zeros((64, 1), f32).at[:F_OUT, 0].set(W_lin[:, 0])
    bl = b_lin.reshape(1, 1)

    offs = (jnp.arange(B, dtype=jnp.int32) % 2 * N).reshape(B, 1, 1)
    adj_row = (adj_matrix.astype(jnp.int32) + offs).reshape(B // 2, 1, E)

    out = pl.pallas_call(
        _phase2_kernel,
        grid=(B // 2,),
        in_specs=[
            pl.BlockSpec((2, N, C), lambda i: (i, 0, 0)),
            pl.BlockSpec((1, 1, E), lambda i: (i, 0, 0)),
            pl.BlockSpec((2, N, M, F_BOND), lambda i: (i, 0, 0, 0)),
            pl.BlockSpec((1, C), lambda i: (0, 0)),
            pl.BlockSpec((1, C), lambda i: (0, 0)),
            pl.BlockSpec((C, 128), lambda i: (0, 0)),
            pl.BlockSpec((C, 128), lambda i: (0, 0)),
            pl.BlockSpec((F_BOND, 128), lambda i: (0, 0)),
            pl.BlockSpec((1, 128), lambda i: (0, 0)),
            pl.BlockSpec((64, 1), lambda i: (0, 0)),
            pl.BlockSpec((1, 1), lambda i: (0, 0)),
        ],
        out_specs=pl.BlockSpec((1, 2, 1), lambda i: (i, 0, 0)),
        out_shape=jax.ShapeDtypeStruct((B // 2, 2, 1), f32),
        compiler_params=pltpu.CompilerParams(
            dimension_semantics=("parallel",)),
    )(h, adj_row, bond, a, c, ws, wn, wbond, bc, wl, bl)

    return out.reshape(B)


# speedup vs baseline: 23.8590x; 1.4466x over previous
import jax
import jax.numpy as jnp
import numpy as np
from jax.experimental import pallas as pl
from jax.experimental.pallas import tpu as pltpu

B, N, M = 1024, 64, 32
F_IN, F_OUT, F_BOND = 128, 25, 16
HALF = N // 2
C = 32
G1 = 64
P1 = B // G1
E = 2 * N * M
T = 2 * N

_SELFHOT = (np.arange(T)[:, None] == (np.arange(E) // M)[None, :]
            ).astype(np.float32)
_SELFHOT_T = np.ascontiguousarray(_SELFHOT.T)


def _softplus(x):
    return jnp.logaddexp(x, 0.0)


def _phase1_kernel(atom_ref, wa_ref, ba_ref, wb_ref, bb_ref, h_ref, st_ref):
    x = atom_ref[...]
    x0 = x[:, :HALF, :].reshape(G1 * HALF, F_IN)
    x1 = x[:, HALF:, :].reshape(G1 * HALF, F_IN)
    h0 = _softplus(jnp.dot(x0, wa_ref[...], preferred_element_type=jnp.float32)
                   + ba_ref[...])
    h1 = _softplus(jnp.dot(x1, wb_ref[...], preferred_element_type=jnp.float32)
                   + bb_ref[...])
    h_ref[...] = jnp.concatenate(
        [h0.reshape(G1, HALF, C), h1.reshape(G1, HALF, C)], axis=1)
    s = (jnp.sum(h0, axis=0) + jnp.sum(h1, axis=0)).reshape(1, 1, C)
    sq = (jnp.sum(h0 * h0, axis=0) + jnp.sum(h1 * h1, axis=0)).reshape(1, 1, C)
    st_ref[...] = jnp.concatenate([s, sq], axis=1)


def _phase2_kernel(h_ref, adj_ref, bond_ref, sh_ref, shT_ref, ws_ref, wn_ref,
                   wb_ref, bc_ref, a_ref, c_ref, wl_ref, bl_ref, out_ref):
    h = h_ref[...].reshape(T, C)
    s = jnp.dot(h, ws_ref[...], preferred_element_type=jnp.float32)
    t = jnp.dot(h, wn_ref[...], preferred_element_type=jnp.float32)
    sT = jnp.transpose(s) + bc_ref[...]
    tT = jnp.transpose(t)
    idx = adj_ref[...].reshape(1, E)
    onehotT = (jax.lax.broadcasted_iota(jnp.int32, (T, E), 0) == idx
               ).astype(jnp.float32)
    gath = jnp.dot(tT, onehotT, preferred_element_type=jnp.float32)
    se = jnp.dot(sT, sh_ref[...], preferred_element_type=jnp.float32)
    zbT = jax.lax.dot_general(wb_ref[...], bond_ref[...].reshape(E, F_BOND),
                              (((0,), (1,)), ((), ())),
                              preferred_element_type=jnp.float32)
    zw = gath + se + zbT
    gT = jax.nn.sigmoid(zw[:C, :])
    cT = _softplus(zw[C:, :])
    msgT = jnp.dot(gT * cT, shT_ref[...],
                   preferred_element_type=jnp.float32)
    hnT = jnp.transpose(h) * a_ref[...] + c_ref[...]
    au = _softplus(hnT + msgT)
    v = jnp.sum(au * wl_ref[...], axis=0, keepdims=True)
    halfmask = (jax.lax.broadcasted_iota(jnp.int32, (T, 2), 0) // N
                == jax.lax.broadcasted_iota(jnp.int32, (T, 2), 1)
                ).astype(jnp.float32) * (1.0 / N)
    e = jnp.dot(v, halfmask, preferred_element_type=jnp.float32)
    out_ref[...] = jnp.maximum(e + bl_ref[...], 0.0).reshape(1, 2, 1)


def kernel(atom, bond, adj_matrix, W_A, b_A, W_B, b_B, gamma, beta,
           W_conv, b_conv, W_lin, b_lin):
    f32 = jnp.float32

    wa = jnp.zeros((F_IN, C), f32).at[:, :F_OUT].set(W_A)
    wb = jnp.zeros((F_IN, C), f32).at[:, :F_OUT].set(W_B)
    ba = jnp.zeros((1, C), f32).at[0, :F_OUT].set(b_A)
    bb = jnp.zeros((1, C), f32).at[0, :F_OUT].set(b_B)

    h, stats = pl.pallas_call(
        _phase1_kernel,
        grid=(P1,),
        in_specs=[
            pl.BlockSpec((G1, N, F_IN), lambda i: (i, 0, 0)),
            pl.BlockSpec((F_IN, C), lambda i: (0, 0)),
            pl.BlockSpec((1, C), lambda i: (0, 0)),
            pl.BlockSpec((F_IN, C), lambda i: (0, 0)),
            pl.BlockSpec((1, C), lambda i: (0, 0)),
        ],
        out_specs=[
            pl.BlockSpec((G1, N, C), lambda i: (i, 0, 0)),
            pl.BlockSpec((1, 2, C), lambda i: (i, 0, 0)),
        ],
        out_shape=[
            jax.ShapeDtypeStruct((B, N, C), f32),
            jax.ShapeDtypeStruct((P1, 2, C), f32),
        ],
        compiler_params=pltpu.CompilerParams(
            dimension_semantics=("parallel",)),
    )(atom, wa, ba, wb, bb)

    cnt = float(B * N)
    mean = jnp.sum(stats[:, 0, :], axis=0) / cnt
    var = jnp.sum(stats[:, 1, :], axis=0) / cnt - mean * mean
    g32 = jnp.zeros((C,), f32).at[:F_OUT].set(gamma)
    be32 = jnp.zeros((C,), f32).at[:F_OUT].set(beta)
    av = g32 * jax.lax.rsqrt(var + 1e-5)
    cv = be32 - mean * av

    def pack64(wx):
        out = jnp.zeros((wx.shape[0], 64), f32)
        out = out.at[:, 0:F_OUT].set(wx[:, :F_OUT])
        return out.at[:, C:C + F_OUT].set(wx[:, F_OUT:])

    wsp = jnp.zeros((C, 64), f32).at[:F_OUT].set(pack64(W_conv[0:F_OUT]))
    wnp = jnp.zeros((C, 64), f32).at[:F_OUT].set(pack64(W_conv[F_OUT:2 * F_OUT]))
    ws2 = av[:, None] * wsp
    wn2 = av[:, None] * wnp
    wb2 = pack64(W_conv[2 * F_OUT:])
    bvec = pack64(b_conv.reshape(1, 2 * F_OUT))[0] + cv @ wsp + cv @ wnp
    bc2 = bvec.reshape(64, 1)
    acol = av.reshape(C, 1)
    ccol = cv.reshape(C, 1)
    wl2 = jnp.zeros((C, 1), f32).at[:F_OUT, 0].set(W_lin[:, 0])
    bl = b_lin.reshape(1, 1)

    offs = (jnp.arange(B, dtype=jnp.int32) % 2 * N).reshape(B, 1, 1)
    adj_row = (adj_matrix.astype(jnp.int32) + offs).reshape(B // 2, 1, E)

    selfhot = jnp.asarray(_SELFHOT)
    selfhotT = jnp.asarray(_SELFHOT_T)

    out = pl.pallas_call(
        _phase2_kernel,
        grid=(B // 2,),
        in_specs=[
            pl.BlockSpec((2, N, C), lambda i: (i, 0, 0)),
            pl.BlockSpec((1, 1, E), lambda i: (i, 0, 0)),
            pl.BlockSpec((2, N, M, F_BOND), lambda i: (i, 0, 0, 0)),
            pl.BlockSpec((T, E), lambda i: (0, 0)),
            pl.BlockSpec((E, T), lambda i: (0, 0)),
            pl.BlockSpec((C, 64), lambda i: (0, 0)),
            pl.BlockSpec((C, 64), lambda i: (0, 0)),
            pl.BlockSpec((F_BOND, 64), lambda i: (0, 0)),
            pl.BlockSpec((64, 1), lambda i: (0, 0)),
            pl.BlockSpec((C, 1), lambda i: (0, 0)),
            pl.BlockSpec((C, 1), lambda i: (0, 0)),
            pl.BlockSpec((C, 1), lambda i: (0, 0)),
            pl.BlockSpec((1, 1), lambda i: (0, 0)),
        ],
        out_specs=pl.BlockSpec((1, 2, 1), lambda i: (i, 0, 0)),
        out_shape=jax.ShapeDtypeStruct((B // 2, 2, 1), f32),
        compiler_params=pltpu.CompilerParams(
            dimension_semantics=("parallel",)),
    )(h, adj_row, bond, selfhot, selfhotT, ws2, wn2, wb2, bc2, acol, ccol,
      wl2, bl)

    return out.reshape(B)


# speedup vs baseline: 35.1128x; 1.4717x over previous
import jax
import jax.numpy as jnp
import numpy as np
from jax.experimental import pallas as pl
from jax.experimental.pallas import tpu as pltpu

B, N, M = 1024, 64, 32
F_IN, F_OUT, F_BOND = 128, 25, 16
HALF = N // 2
C = 32
G1 = 64
P1 = B // G1
E = 2 * N * M
T = 2 * N
PAIRS = 8

def _softplus(x):
    return jnp.logaddexp(x, 0.0)


def _phase1_kernel(atom_ref, wa_ref, ba_ref, wb_ref, bb_ref, h_ref, st_ref):
    x = atom_ref[...]
    x0 = x[:, :HALF, :].reshape(G1 * HALF, F_IN)
    x1 = x[:, HALF:, :].reshape(G1 * HALF, F_IN)
    h0 = _softplus(jnp.dot(x0, wa_ref[...], preferred_element_type=jnp.float32)
                   + ba_ref[...])
    h1 = _softplus(jnp.dot(x1, wb_ref[...], preferred_element_type=jnp.float32)
                   + bb_ref[...])
    h_ref[...] = jnp.concatenate(
        [h0.reshape(G1, HALF, C), h1.reshape(G1, HALF, C)], axis=1)
    s = (jnp.sum(h0, axis=0) + jnp.sum(h1, axis=0)).reshape(1, 1, C)
    sq = (jnp.sum(h0 * h0, axis=0) + jnp.sum(h1 * h1, axis=0)).reshape(1, 1, C)
    st_ref[...] = jnp.concatenate([s, sq], axis=1)


def _phase2_kernel(h_ref, adj_ref, bond_ref, ws_ref, wn_ref,
                   wb_ref, bc_ref, a_ref, c_ref, wl_ref, bl_ref, out_ref):
    h_all = h_ref[...].reshape(PAIRS * T, C)
    bond_all = bond_ref[...].reshape(2 * PAIRS * N, M * F_BOND)
    halfmask = (jax.lax.broadcasted_iota(jnp.int32, (T, 2), 0) // N
                == jax.lax.broadcasted_iota(jnp.int32, (T, 2), 1)
                ).astype(jnp.float32) * (1.0 / N)
    es = []
    for p in range(PAIRS):
        h = h_all[p * T:(p + 1) * T, :]
        s = jnp.dot(h, ws_ref[...], preferred_element_type=jnp.float32)
        t = jnp.dot(h, wn_ref[...], preferred_element_type=jnp.float32)
        sT = jnp.transpose(s) + bc_ref[...]
        tT = jnp.transpose(t)
        idx = adj_ref[0, p, :].reshape(1, E)
        onehotT = (jax.lax.broadcasted_iota(jnp.int32, (T, E), 0) == idx
                   ).astype(jnp.float32)
        gath = jnp.dot(tT, onehotT, preferred_element_type=jnp.float32)
        se = jnp.concatenate([sT] * M, axis=1)
        blk = bond_all[p * T:(p + 1) * T, :]
        rhs16 = jnp.concatenate(
            [jnp.transpose(blk[:, m * F_BOND:(m + 1) * F_BOND])
             for m in range(M)], axis=1)
        zbT = jax.lax.dot_general(
            wb_ref[...], rhs16, (((0,), (0,)), ((), ())),
            preferred_element_type=jnp.float32)
        zw = gath + se + zbT
        gT = jax.nn.sigmoid(zw[:C, :])
        cT = _softplus(zw[C:, :])
        q = gT * cT
        w = E
        while w > T:
            w //= 2
            q = q[:, :w] + q[:, w:2 * w]
        msgT = q
        hnT = jnp.transpose(h) * a_ref[...] + c_ref[...]
        au = _softplus(hnT + msgT)
        v = jnp.sum(au * wl_ref[...], axis=0, keepdims=True)
        es.append(jnp.dot(v, halfmask, preferred_element_type=jnp.float32))
    e = jnp.concatenate(es, axis=1)
    out_ref[...] = jnp.maximum(e + bl_ref[...], 0.0).reshape(1, 2 * PAIRS, 1)


def kernel(atom, bond, adj_matrix, W_A, b_A, W_B, b_B, gamma, beta,
           W_conv, b_conv, W_lin, b_lin):
    f32 = jnp.float32

    wa = jnp.zeros((F_IN, C), f32).at[:, :F_OUT].set(W_A)
    wb = jnp.zeros((F_IN, C), f32).at[:, :F_OUT].set(W_B)
    ba = jnp.zeros((1, C), f32).at[0, :F_OUT].set(b_A)
    bb = jnp.zeros((1, C), f32).at[0, :F_OUT].set(b_B)

    h, stats = pl.pallas_call(
        _phase1_kernel,
        grid=(P1,),
        in_specs=[
            pl.BlockSpec((G1, N, F_IN), lambda i: (i, 0, 0)),
            pl.BlockSpec((F_IN, C), lambda i: (0, 0)),
            pl.BlockSpec((1, C), lambda i: (0, 0)),
            pl.BlockSpec((F_IN, C), lambda i: (0, 0)),
            pl.BlockSpec((1, C), lambda i: (0, 0)),
        ],
        out_specs=[
            pl.BlockSpec((G1, N, C), lambda i: (i, 0, 0)),
            pl.BlockSpec((1, 2, C), lambda i: (i, 0, 0)),
        ],
        out_shape=[
            jax.ShapeDtypeStruct((B, N, C), f32),
            jax.ShapeDtypeStruct((P1, 2, C), f32),
        ],
        compiler_params=pltpu.CompilerParams(
            dimension_semantics=("parallel",)),
    )(atom, wa, ba, wb, bb)

    cnt = float(B * N)
    mean = jnp.sum(stats[:, 0, :], axis=0) / cnt
    var = jnp.sum(stats[:, 1, :], axis=0) / cnt - mean * mean
    g32 = jnp.zeros((C,), f32).at[:F_OUT].set(gamma)
    be32 = jnp.zeros((C,), f32).at[:F_OUT].set(beta)
    av = g32 * jax.lax.rsqrt(var + 1e-5)
    cv = be32 - mean * av

    def pack64(wx):
        out = jnp.zeros((wx.shape[0], 64), f32)
        out = out.at[:, 0:F_OUT].set(wx[:, :F_OUT])
        return out.at[:, C:C + F_OUT].set(wx[:, F_OUT:])

    wsp = jnp.zeros((C, 64), f32).at[:F_OUT].set(pack64(W_conv[0:F_OUT]))
    wnp = jnp.zeros((C, 64), f32).at[:F_OUT].set(pack64(W_conv[F_OUT:2 * F_OUT]))
    ws2 = av[:, None] * wsp
    wn2 = av[:, None] * wnp
    wb2 = pack64(W_conv[2 * F_OUT:])
    bvec = pack64(b_conv.reshape(1, 2 * F_OUT))[0] + cv @ wsp + cv @ wnp
    bc2 = bvec.reshape(64, 1)
    acol = av.reshape(C, 1)
    ccol = cv.reshape(C, 1)
    wl2 = jnp.zeros((C, 1), f32).at[:F_OUT, 0].set(W_lin[:, 0])
    bl = b_lin.reshape(1, 1)

    offs = (jnp.arange(B, dtype=jnp.int32) % 2 * N).reshape(B, 1, 1)
    adjg = (adj_matrix.astype(jnp.int32) + offs).reshape(B // 2, 2, N, M)
    adj_row = jnp.transpose(adjg, (0, 3, 1, 2)).reshape(B // (2 * PAIRS),
                                                        PAIRS, E)
    bond_r = bond.reshape(B, N, M * F_BOND)

    out = pl.pallas_call(
        _phase2_kernel,
        grid=(B // (2 * PAIRS),),
        in_specs=[
            pl.BlockSpec((2 * PAIRS, N, C), lambda i: (i, 0, 0)),
            pl.BlockSpec((1, PAIRS, E), lambda i: (i, 0, 0)),
            pl.BlockSpec((2 * PAIRS, N, M * F_BOND), lambda i: (i, 0, 0)),
            pl.BlockSpec((C, 64), lambda i: (0, 0)),
            pl.BlockSpec((C, 64), lambda i: (0, 0)),
            pl.BlockSpec((F_BOND, 64), lambda i: (0, 0)),
            pl.BlockSpec((64, 1), lambda i: (0, 0)),
            pl.BlockSpec((C, 1), lambda i: (0, 0)),
            pl.BlockSpec((C, 1), lambda i: (0, 0)),
            pl.BlockSpec((C, 1), lambda i: (0, 0)),
            pl.BlockSpec((1, 1), lambda i: (0, 0)),
        ],
        out_specs=pl.BlockSpec((1, 2 * PAIRS, 1), lambda i: (i, 0, 0)),
        out_shape=jax.ShapeDtypeStruct((B // (2 * PAIRS), 2 * PAIRS, 1), f32),
        compiler_params=pltpu.CompilerParams(
            dimension_semantics=("parallel",)),
    )(h, adj_row, bond_r, ws2, wn2, wb2, bc2, acol, ccol, wl2, bl)

    return out.reshape(B)


# speedup vs baseline: 36.4865x; 1.0391x over previous
import jax
import jax.numpy as jnp
import numpy as np
from jax.experimental import pallas as pl
from jax.experimental.pallas import tpu as pltpu

B, N, M = 1024, 64, 32
F_IN, F_OUT, F_BOND = 128, 25, 16
HALF = N // 2
C = 32
G1 = 64
P1 = B // G1
E = 2 * N * M
T = 2 * N
PAIRS = 16

def _softplus(x):
    return jnp.logaddexp(x, 0.0)


def _phase1_kernel(atom_ref, wa_ref, ba_ref, wb_ref, bb_ref, h_ref, st_ref):
    x = atom_ref[...]
    x0 = x[:, :HALF, :].reshape(G1 * HALF, F_IN)
    x1 = x[:, HALF:, :].reshape(G1 * HALF, F_IN)
    h0 = _softplus(jnp.dot(x0, wa_ref[...], preferred_element_type=jnp.float32)
                   + ba_ref[...])
    h1 = _softplus(jnp.dot(x1, wb_ref[...], preferred_element_type=jnp.float32)
                   + bb_ref[...])
    h_ref[...] = jnp.concatenate(
        [h0.reshape(G1, HALF, C), h1.reshape(G1, HALF, C)], axis=1)
    s = (jnp.sum(h0, axis=0) + jnp.sum(h1, axis=0)).reshape(1, 1, C)
    sq = (jnp.sum(h0 * h0, axis=0) + jnp.sum(h1 * h1, axis=0)).reshape(1, 1, C)
    st_ref[...] = jnp.concatenate([s, sq], axis=1)


def _phase2_kernel(h_ref, adj_ref, bond_ref, ws_ref, wn_ref,
                   wb_ref, bc_ref, a_ref, c_ref, wl_ref, bl_ref, out_ref):
    h_all = h_ref[...].reshape(PAIRS * T, C)
    bond_all = bond_ref[...].reshape(2 * PAIRS * N, M * F_BOND)
    halfmask = (jax.lax.broadcasted_iota(jnp.int32, (T, 2), 0) // N
                == jax.lax.broadcasted_iota(jnp.int32, (T, 2), 1)
                ).astype(jnp.float32) * (1.0 / N)
    es = []
    for p in range(PAIRS):
        h = h_all[p * T:(p + 1) * T, :]
        s = jnp.dot(h, ws_ref[...], preferred_element_type=jnp.float32)
        t = jnp.dot(h, wn_ref[...], preferred_element_type=jnp.float32)
        sT = jnp.transpose(s) + bc_ref[...]
        tT = jnp.transpose(t)
        idx = adj_ref[0, p, :].reshape(1, E)
        onehotT = (jax.lax.broadcasted_iota(jnp.int32, (T, E), 0) == idx
                   ).astype(jnp.float32)
        gath = jnp.dot(tT, onehotT, preferred_element_type=jnp.float32)
        se = jnp.concatenate([sT] * M, axis=1)
        blk = bond_all[p * T:(p + 1) * T, :]
        rhs16 = jnp.concatenate(
            [jnp.transpose(blk[:, m * F_BOND:(m + 1) * F_BOND])
             for m in range(M)], axis=1)
        zbT = jax.lax.dot_general(
            wb_ref[...], rhs16, (((0,), (0,)), ((), ())),
            preferred_element_type=jnp.float32)
        zw = gath + se + zbT
        gT = jax.nn.sigmoid(zw[:C, :])
        cT = _softplus(zw[C:, :])
        q = gT * cT
        w = E
        while w > T:
            w //= 2
            q = q[:, :w] + q[:, w:2 * w]
        msgT = q
        hnT = jnp.transpose(h) * a_ref[...] + c_ref[...]
        au = _softplus(hnT + msgT)
        v = jnp.sum(au * wl_ref[...], axis=0, keepdims=True)
        es.append(jnp.dot(v, halfmask, preferred_element_type=jnp.float32))
    e = jnp.concatenate(es, axis=1)
    out_ref[...] = jnp.maximum(e + bl_ref[...], 0.0).reshape(1, 2 * PAIRS, 1)


def kernel(atom, bond, adj_matrix, W_A, b_A, W_B, b_B, gamma, beta,
           W_conv, b_conv, W_lin, b_lin):
    f32 = jnp.float32

    wa = jnp.zeros((F_IN, C), f32).at[:, :F_OUT].set(W_A)
    wb = jnp.zeros((F_IN, C), f32).at[:, :F_OUT].set(W_B)
    ba = jnp.zeros((1, C), f32).at[0, :F_OUT].set(b_A)
    bb = jnp.zeros((1, C), f32).at[0, :F_OUT].set(b_B)

    h, stats = pl.pallas_call(
        _phase1_kernel,
        grid=(P1,),
        in_specs=[
            pl.BlockSpec((G1, N, F_IN), lambda i: (i, 0, 0)),
            pl.BlockSpec((F_IN, C), lambda i: (0, 0)),
            pl.BlockSpec((1, C), lambda i: (0, 0)),
            pl.BlockSpec((F_IN, C), lambda i: (0, 0)),
            pl.BlockSpec((1, C), lambda i: (0, 0)),
        ],
        out_specs=[
            pl.BlockSpec((G1, N, C), lambda i: (i, 0, 0)),
            pl.BlockSpec((1, 2, C), lambda i: (i, 0, 0)),
        ],
        out_shape=[
            jax.ShapeDtypeStruct((B, N, C), f32),
            jax.ShapeDtypeStruct((P1, 2, C), f32),
        ],
        compiler_params=pltpu.CompilerParams(
            dimension_semantics=("parallel",)),
    )(atom, wa, ba, wb, bb)

    cnt = float(B * N)
    mean = jnp.sum(stats[:, 0, :], axis=0) / cnt
    var = jnp.sum(stats[:, 1, :], axis=0) / cnt - mean * mean
    g32 = jnp.zeros((C,), f32).at[:F_OUT].set(gamma)
    be32 = jnp.zeros((C,), f32).at[:F_OUT].set(beta)
    av = g32 * jax.lax.rsqrt(var + 1e-5)
    cv = be32 - mean * av

    def pack64(wx):
        out = jnp.zeros((wx.shape[0], 64), f32)
        out = out.at[:, 0:F_OUT].set(wx[:, :F_OUT])
        return out.at[:, C:C + F_OUT].set(wx[:, F_OUT:])

    wsp = jnp.zeros((C, 64), f32).at[:F_OUT].set(pack64(W_conv[0:F_OUT]))
    wnp = jnp.zeros((C, 64), f32).at[:F_OUT].set(pack64(W_conv[F_OUT:2 * F_OUT]))
    ws2 = av[:, None] * wsp
    wn2 = av[:, None] * wnp
    wb2 = pack64(W_conv[2 * F_OUT:])
    bvec = pack64(b_conv.reshape(1, 2 * F_OUT))[0] + cv @ wsp + cv @ wnp
    bc2 = bvec.reshape(64, 1)
    acol = av.reshape(C, 1)
    ccol = cv.reshape(C, 1)
    wl2 = jnp.zeros((C, 1), f32).at[:F_OUT, 0].set(W_lin[:, 0])
    bl = b_lin.reshape(1, 1)

    offs = (jnp.arange(B, dtype=jnp.int32) % 2 * N).reshape(B, 1, 1)
    adjg = (adj_matrix.astype(jnp.int32) + offs).reshape(B // 2, 2, N, M)
    adj_row = jnp.transpose(adjg, (0, 3, 1, 2)).reshape(B // (2 * PAIRS),
                                                        PAIRS, E)
    bond_r = bond.reshape(B, N, M * F_BOND)

    out = pl.pallas_call(
        _phase2_kernel,
        grid=(B // (2 * PAIRS),),
        in_specs=[
            pl.BlockSpec((2 * PAIRS, N, C), lambda i: (i, 0, 0)),
            pl.BlockSpec((1, PAIRS, E), lambda i: (i, 0, 0)),
            pl.BlockSpec((2 * PAIRS, N, M * F_BOND), lambda i: (i, 0, 0)),
            pl.BlockSpec((C, 64), lambda i: (0, 0)),
            pl.BlockSpec((C, 64), lambda i: (0, 0)),
            pl.BlockSpec((F_BOND, 64), lambda i: (0, 0)),
            pl.BlockSpec((64, 1), lambda i: (0, 0)),
            pl.BlockSpec((C, 1), lambda i: (0, 0)),
            pl.BlockSpec((C, 1), lambda i: (0, 0)),
            pl.BlockSpec((C, 1), lambda i: (0, 0)),
            pl.BlockSpec((1, 1), lambda i: (0, 0)),
        ],
        out_specs=pl.BlockSpec((1, 2 * PAIRS, 1), lambda i: (i, 0, 0)),
        out_shape=jax.ShapeDtypeStruct((B // (2 * PAIRS), 2 * PAIRS, 1), f32),
        compiler_params=pltpu.CompilerParams(
            dimension_semantics=("parallel",)),
    )(h, adj_row, bond_r, ws2, wn2, wb2, bc2, acol, ccol, wl2, bl)

    return out.reshape(B)
